# Initial kernel scaffold; baseline (speedup 1.0000x reference)
import jax
import jax.numpy as jnp
from jax.experimental import pallas as pl


def kernel(indices, table):
    raise NotImplementedError("write your pallas kernel here")


# speedup vs baseline: 3.0264x; 3.0264x over previous
import functools

import jax
import jax.numpy as jnp
from jax import lax
from jax.experimental import pallas as pl
from jax.experimental.pallas import tpu as pltpu
from jax.experimental.pallas import tpu_sc as plsc

NUM_CORES = 2
NUM_SUBCORES = 16
NUM_WORKERS = NUM_CORES * NUM_SUBCORES
CHUNK = 128


def _gather_fn(B, D):
    bpw = B // NUM_WORKERS
    nchunks = bpw // CHUNK
    mesh = plsc.VectorSubcoreMesh(core_axis_name="c", subcore_axis_name="s")

    @functools.partial(
        pl.kernel,
        out_type=jax.ShapeDtypeStruct((B, D), jnp.float32),
        mesh=mesh,
        scratch_types=[
            pltpu.VMEM((bpw,), jnp.int32),
            pltpu.VMEM((CHUNK, D), jnp.float32),
            pltpu.SemaphoreType.DMA,
        ],
    )
    def gather_kernel(idx_hbm, table_hbm, out_hbm, idx_v, buf, sem):
        wid = lax.axis_index("s") * NUM_CORES + lax.axis_index("c")
        base = wid * bpw
        pltpu.sync_copy(idx_hbm.at[pl.ds(base, bpw)], idx_v)

        def step(k, carry):
            pltpu.async_copy(
                table_hbm.at[idx_v.at[pl.ds(k * CHUNK, CHUNK)]], buf, sem
            ).wait()
            pltpu.sync_copy(buf, out_hbm.at[pl.ds(base + k * CHUNK, CHUNK)])
            return carry

        lax.fori_loop(0, nchunks, step, 0)

    return gather_kernel


@jax.jit
def kernel(indices, table):
    B = indices.size
    V, D = table.shape
    idx_flat = indices.reshape(B).astype(jnp.int32)
    out = _gather_fn(B, D)(idx_flat, table)
    return out.reshape(indices.shape + (D,))


# speedup vs baseline: 3.2956x; 1.0890x over previous
import functools

import jax
import jax.numpy as jnp
from jax import lax
from jax.experimental import pallas as pl
from jax.experimental.pallas import tpu as pltpu
from jax.experimental.pallas import tpu_sc as plsc

NUM_CORES = 2
NUM_SUBCORES = 16
NUM_WORKERS = NUM_CORES * NUM_SUBCORES
CHUNK = 64
NBUF = 2


def _gather_fn(B, D):
    bpw = B // NUM_WORKERS
    nchunks = bpw // CHUNK
    npairs = nchunks // NBUF
    mesh = plsc.VectorSubcoreMesh(core_axis_name="c", subcore_axis_name="s")

    @functools.partial(
        pl.kernel,
        out_type=jax.ShapeDtypeStruct((B, D), jnp.float32),
        mesh=mesh,
        scratch_types=[
            pltpu.VMEM((bpw,), jnp.int32),
            [pltpu.VMEM((CHUNK, D), jnp.float32) for _ in range(NBUF)],
            [pltpu.SemaphoreType.DMA for _ in range(NBUF)],
            [pltpu.SemaphoreType.DMA for _ in range(NBUF)],
        ],
    )
    def gather_kernel(idx_hbm, table_hbm, out_hbm, idx_v, bufs, gsems, osems):
        wid = lax.axis_index("s") * NUM_CORES + lax.axis_index("c")
        base = wid * bpw
        pltpu.sync_copy(idx_hbm.at[pl.ds(base, bpw)], idx_v)

        def g_copy(k, j):
            return pltpu.make_async_copy(
                table_hbm.at[idx_v.at[pl.ds(k * CHUNK, CHUNK)]],
                bufs[j], gsems[j],
            )

        def w_copy(k, j):
            return pltpu.make_async_copy(
                bufs[j], out_hbm.at[pl.ds(base + k * CHUNK, CHUNK)], osems[j]
            )

        for j in range(NBUF):
            g_copy(j, j).start()

        def pair(p, carry):
            for j in range(NBUF):
                k = p * NBUF + j
                g_copy(k, j).wait()
                w_copy(k, j).start()
                w_copy(k, j).wait()
                g_copy(k + NBUF, j).start()
            return carry

        lax.fori_loop(0, npairs - 1, pair, 0)

        for j in range(NBUF):
            k = (npairs - 1) * NBUF + j
            g_copy(k, j).wait()
            w_copy(k, j).start()
        for j in range(NBUF):
            k = (npairs - 1) * NBUF + j
            w_copy(k, j).wait()

    return gather_kernel


@jax.jit
def kernel(indices, table):
    B = indices.size
    V, D = table.shape
    idx_flat = indices.reshape(B).astype(jnp.int32)
    out = _gather_fn(B, D)(idx_flat, table)
    return out.reshape(indices.shape + (D,))


# speedup vs baseline: 3.3092x; 1.0041x over previous
import functools

import jax
import jax.numpy as jnp
from jax import lax
from jax.experimental import pallas as pl
from jax.experimental.pallas import tpu as pltpu
from jax.experimental.pallas import tpu_sc as plsc

NUM_CORES = 2
NUM_SUBCORES = 16
NUM_WORKERS = NUM_CORES * NUM_SUBCORES
CHUNK = 64
NBUF = 3


def _gather_fn(B, D):
    bpw = B // NUM_WORKERS
    nchunks = bpw // CHUNK
    npairs = nchunks // NBUF
    mesh = plsc.VectorSubcoreMesh(core_axis_name="c", subcore_axis_name="s")

    @functools.partial(
        pl.kernel,
        out_type=jax.ShapeDtypeStruct((B, D), jnp.float32),
        mesh=mesh,
        scratch_types=[
            pltpu.VMEM((bpw,), jnp.int32),
            [pltpu.VMEM((CHUNK, D), jnp.float32) for _ in range(NBUF)],
            [pltpu.SemaphoreType.DMA for _ in range(NBUF)],
            [pltpu.SemaphoreType.DMA for _ in range(NBUF)],
        ],
    )
    def gather_kernel(idx_hbm, table_hbm, out_hbm, idx_v, bufs, gsems, osems):
        wid = lax.axis_index("s") * NUM_CORES + lax.axis_index("c")
        base = wid * bpw
        pltpu.sync_copy(idx_hbm.at[pl.ds(base, bpw)], idx_v)

        def g_copy(k, j):
            return pltpu.make_async_copy(
                table_hbm.at[idx_v.at[pl.ds(k * CHUNK, CHUNK)]],
                bufs[j], gsems[j],
            )

        def w_copy(k, j):
            return pltpu.make_async_copy(
                bufs[j], out_hbm.at[pl.ds(base + k * CHUNK, CHUNK)], osems[j]
            )

        for j in range(NBUF):
            g_copy(j, j).start()

        def step(k, j):
            g_copy(k, j).wait()
            w_copy(k, j).start()
            w_copy(k, j).wait()
            g_copy(k + NBUF, j).start()

        full_rounds = (nchunks - NBUF) // NBUF
        rem = (nchunks - NBUF) % NBUF

        def round_body(p, carry):
            for j in range(NBUF):
                step(p * NBUF + j, j)
            return carry

        lax.fori_loop(0, full_rounds, round_body, 0)

        for t in range(rem):
            k = full_rounds * NBUF + t
            step(k, k % NBUF)
        for t in range(NBUF):
            k = nchunks - NBUF + t
            g_copy(k, k % NBUF).wait()
            w_copy(k, k % NBUF).start()
        for t in range(NBUF):
            k = nchunks - NBUF + t
            w_copy(k, k % NBUF).wait()

    return gather_kernel


@jax.jit
def kernel(indices, table):
    B = indices.size
    V, D = table.shape
    idx_flat = indices.reshape(B).astype(jnp.int32)
    out = _gather_fn(B, D)(idx_flat, table)
    return out.reshape(indices.shape + (D,))


# speedup vs baseline: 3.3158x; 1.0020x over previous
import functools

import jax
import jax.numpy as jnp
from jax import lax
from jax.experimental import pallas as pl
from jax.experimental.pallas import tpu as pltpu
from jax.experimental.pallas import tpu_sc as plsc

NUM_CORES = 2
NUM_SUBCORES = 16
NUM_WORKERS = NUM_CORES * NUM_SUBCORES
CHUNK = 32
NBUF = 6


def _gather_fn(B, D):
    bpw = B // NUM_WORKERS
    nchunks = bpw // CHUNK
    npairs = nchunks // NBUF
    mesh = plsc.VectorSubcoreMesh(core_axis_name="c", subcore_axis_name="s")

    @functools.partial(
        pl.kernel,
        out_type=jax.ShapeDtypeStruct((B, D), jnp.float32),
        mesh=mesh,
        scratch_types=[
            pltpu.VMEM((bpw,), jnp.int32),
            [pltpu.VMEM((CHUNK, D), jnp.float32) for _ in range(NBUF)],
            [pltpu.SemaphoreType.DMA for _ in range(NBUF)],
            [pltpu.SemaphoreType.DMA for _ in range(NBUF)],
        ],
    )
    def gather_kernel(idx_hbm, table_hbm, out_hbm, idx_v, bufs, gsems, osems):
        wid = lax.axis_index("s") * NUM_CORES + lax.axis_index("c")
        base = wid * bpw
        pltpu.sync_copy(idx_hbm.at[pl.ds(base, bpw)], idx_v)

        def g_copy(k, j):
            return pltpu.make_async_copy(
                table_hbm.at[idx_v.at[pl.ds(k * CHUNK, CHUNK)]],
                bufs[j], gsems[j],
            )

        def w_copy(k, j):
            return pltpu.make_async_copy(
                bufs[j], out_hbm.at[pl.ds(base + k * CHUNK, CHUNK)], osems[j]
            )

        for j in range(NBUF):
            g_copy(j, j).start()

        def step(k, j):
            g_copy(k, j).wait()
            w_copy(k, j).start()
            w_copy(k, j).wait()
            g_copy(k + NBUF, j).start()

        full_rounds = (nchunks - NBUF) // NBUF
        rem = (nchunks - NBUF) % NBUF

        def round_body(p, carry):
            for j in range(NBUF):
                step(p * NBUF + j, j)
            return carry

        lax.fori_loop(0, full_rounds, round_body, 0)

        for t in range(rem):
            k = full_rounds * NBUF + t
            step(k, k % NBUF)
        for t in range(NBUF):
            k = nchunks - NBUF + t
            g_copy(k, k % NBUF).wait()
            w_copy(k, k % NBUF).start()
        for t in range(NBUF):
            k = nchunks - NBUF + t
            w_copy(k, k % NBUF).wait()

    return gather_kernel


@jax.jit
def kernel(indices, table):
    B = indices.size
    V, D = table.shape
    idx_flat = indices.reshape(B).astype(jnp.int32)
    out = _gather_fn(B, D)(idx_flat, table)
    return out.reshape(indices.shape + (D,))
